# baseline (speedup 1.0000x reference)
import jax
import jax.numpy as jnp
from jax.experimental import pallas as pl
from jax.experimental.pallas import tpu as pltpu

_BOUND = 512.0
_NBLK = 2


def _fsq_body(z_ref, out_ref, loss_ref, acc_ref):
    i = pl.program_id(0)
    z = z_ref[...]
    qz = jnp.round(_BOUND * jnp.tanh(z))
    delta = qz - z
    out_ref[...] = qz
    part = jnp.sum(delta * delta, axis=0)

    @pl.when(i == 0)
    def _init():
        acc_ref[...] = part

    @pl.when(i > 0)
    def _accum():
        acc_ref[...] += part

    @pl.when(i == _NBLK - 1)
    def _finish():
        loss_ref[0, 0] = jnp.sum(acc_ref[...])


def kernel(z, codebook):
    del codebook
    b, s, d = z.shape
    n = z.size
    zt = jnp.transpose(z, (0, 2, 1))
    blk = b // _NBLK
    out_t, loss = pl.pallas_call(
        _fsq_body,
        grid=(_NBLK,),
        compiler_params=pltpu.CompilerParams(
            vmem_limit_bytes=57 * 1024 * 1024,
        ),
        in_specs=[pl.BlockSpec((blk, d, s), lambda i: (i, 0, 0))],
        out_specs=[
            pl.BlockSpec((blk, d, s), lambda i: (i, 0, 0)),
            pl.BlockSpec(memory_space=pltpu.SMEM),
        ],
        out_shape=[
            jax.ShapeDtypeStruct((b, d, s), z.dtype),
            jax.ShapeDtypeStruct((1, 1), jnp.float32),
        ],
        scratch_shapes=[pltpu.VMEM((d, s), jnp.float32)],
    )(zt)
    total_loss = loss[0, 0] * (2.0 / n)
    return (jnp.transpose(out_t, (0, 2, 1)), total_loss)
